# baseline (speedup 1.0000x reference)
import functools

import jax
import jax.numpy as jnp
from jax import lax
from jax.experimental import pallas as pl
from jax.experimental.pallas import tpu as pltpu
from jax.experimental.pallas import tpu_sc as plsc

N_NODES = 100000
N_EDGES = 1600000
GW = 16
NG = 4
K1 = 1000
K2 = 1000
ROWS_PER_TILE = 6256
N_PAD = ROWS_PER_TILE * 16
LAST_ROWS = N_NODES - 15 * ROWS_PER_TILE


def _flush_acc_slice(acc, out, s):
    @pl.when(s < 15)
    def _():
        sl = pl.ds(s * ROWS_PER_TILE, ROWS_PER_TILE)
        pltpu.sync_copy(acc.at[sl], out.at[sl])

    @pl.when(s == 15)
    def _():
        sl = pl.ds(15 * ROWS_PER_TILE, LAST_ROWS)
        pltpu.sync_copy(acc.at[sl], out.at[sl])


def _zero_acc_slice(zer, acc, s):
    sl = pl.ds(s * ROWS_PER_TILE, ROWS_PER_TILE)
    pltpu.sync_copy(zer.at[sl], acc.at[sl])


def _edge_chunks(src, dst, x_hbm, bufA, bufB, acc, base, k, n_chunks):
    iss, idd, rw, g = bufA

    def body(i, carry):
        off = base + i * k
        pltpu.sync_copy(src.at[pl.ds(off, k)], iss)
        pltpu.sync_copy(dst.at[pl.ds(off, k)], idd)
        pltpu.async_copy(x_hbm.at[iss], rw, g).wait()
        pltpu.sync_copy(rw, acc.at[idd], add=True)
        return carry

    lax.fori_loop(0, n_chunks, body, 0)


_MESH = plsc.VectorSubcoreMesh(core_axis_name="c", subcore_axis_name="s")


@functools.partial(
    pl.kernel,
    mesh=_MESH,
    compiler_params=pltpu.CompilerParams(use_tc_tiling_on_sc=False),
    out_type=jax.ShapeDtypeStruct((2, N_NODES), jnp.float32),
    scratch_types=[
        pltpu.VMEM((K1,), jnp.int32),
        pltpu.VMEM((K1,), jnp.int32),
        pltpu.VMEM((K1,), jnp.float32),
        pltpu.VMEM((K1,), jnp.int32),
        pltpu.VMEM((K1,), jnp.int32),
        pltpu.VMEM((K1,), jnp.float32),
        pltpu.VMEM_SHARED((N_PAD,), jnp.float32),
        pltpu.SemaphoreType.DMA,
        pltpu.SemaphoreType.DMA,
    ],
)
def _sc_spmv1(src, dst, feats, zer1, pout,
              isA, idA, rwA, isB, idB, rwB, acc, gA, gB):
    c = lax.axis_index("c")
    s = lax.axis_index("s")
    _zero_acc_slice(zer1, acc, s)
    plsc.subcore_barrier()
    tid = c * 16 + s
    e_per_tile = N_EDGES // 32
    _edge_chunks(src, dst, feats, (isA, idA, rwA, gA),
                 (isB, idB, rwB, gB), acc, tid * e_per_tile, K1,
                 e_per_tile // K1)
    plsc.subcore_barrier()
    _flush_acc_slice(acc, pout.at[c], s)


@functools.partial(
    pl.kernel,
    mesh=_MESH,
    compiler_params=pltpu.CompilerParams(use_tc_tiling_on_sc=False),
    out_type=jax.ShapeDtypeStruct((NG, N_NODES, GW), jnp.float32),
    scratch_types=[
        pltpu.VMEM((K2,), jnp.int32),
        pltpu.VMEM((K2,), jnp.int32),
        pltpu.VMEM((K2, GW), jnp.float32),
        pltpu.VMEM((K2,), jnp.int32),
        pltpu.VMEM((K2,), jnp.int32),
        pltpu.VMEM((K2, GW), jnp.float32),
        pltpu.VMEM_SHARED((N_PAD, GW), jnp.float32),
        pltpu.SemaphoreType.DMA,
        pltpu.SemaphoreType.DMA,
    ],
)
def _sc_spmv64(src, dst, xin, zer, ost,
               isA, idA, rwA, isB, idB, rwB, acc, gA, gB):
    c = lax.axis_index("c")
    s = lax.axis_index("s")
    e_per_tile = N_EDGES // 16
    for p in range(2):
        g = 2 * p + c
        _zero_acc_slice(zer, acc, s)
        plsc.subcore_barrier()
        _edge_chunks(src, dst, xin.at[g], (isA, idA, rwA, gA),
                     (isB, idB, rwB, gB), acc, s * e_per_tile, K2,
                     e_per_tile // K2)
        plsc.subcore_barrier()
        _flush_acc_slice(acc, ost.at[g], s)
        plsc.subcore_barrier()


NB = 2000


def _tc1_body(p, w1, b1, xfull, gout):
    a = p[0] + p[1]
    y = jnp.maximum(a * w1[...] + b1[...], 0.0)
    xfull[...] = y
    for i in range(NG):
        gout[i] = y[:, i * GW:(i + 1) * GW]


def _tc2_body(xf, ain, w2, b2, gout):
    acat = jnp.concatenate([ain[i] for i in range(NG)], axis=1)
    y = jnp.dot(acat, w2[...], preferred_element_type=jnp.float32)
    y = jnp.maximum(xf[...] + y + b2[...], 0.0)
    for i in range(NG):
        gout[i] = y[:, i * GW:(i + 1) * GW]


def _tc3_body(ain, w3, b3, o):
    acat = jnp.concatenate([ain[i] for i in range(NG)], axis=1)
    o[...] = jnp.dot(acat, w3[...], preferred_element_type=jnp.float32) + b3[...]


def _row_spec(cols):
    return pl.BlockSpec((NB, cols), lambda i: (i, 0))


def _full_spec(r, cols):
    return pl.BlockSpec((r, cols), lambda i: (0, 0))


_GRID = (N_NODES // NB,)

_P_SPEC = pl.BlockSpec((2, NB, 1), lambda i: (0, i, 0))
_G_SPEC = pl.BlockSpec((NG, NB, GW), lambda i: (0, i, 0))

_tc1 = pl.pallas_call(
    _tc1_body,
    grid=_GRID,
    in_specs=[_P_SPEC, _full_spec(1, 64), _full_spec(1, 64)],
    out_specs=[_row_spec(64), _G_SPEC],
    out_shape=[jax.ShapeDtypeStruct((N_NODES, 64), jnp.float32),
               jax.ShapeDtypeStruct((NG, N_NODES, GW), jnp.float32)],
)

_tc2 = pl.pallas_call(
    _tc2_body,
    grid=_GRID,
    in_specs=[_row_spec(64), _G_SPEC, _full_spec(64, 64), _full_spec(1, 64)],
    out_specs=_G_SPEC,
    out_shape=jax.ShapeDtypeStruct((NG, N_NODES, GW), jnp.float32),
)

_tc3 = pl.pallas_call(
    _tc3_body,
    grid=_GRID,
    in_specs=[_G_SPEC, _full_spec(64, 128), _full_spec(1, 128)],
    out_specs=_row_spec(128),
    out_shape=jax.ShapeDtypeStruct((N_NODES, 128), jnp.float32),
)


def kernel(features, edge_index, W1, b1, W2, b2, W3, b3):
    src = edge_index[0].astype(jnp.int32)
    dst = edge_index[1].astype(jnp.int32)
    zer = jnp.zeros((N_PAD, GW), jnp.float32)
    zer1 = jnp.zeros((N_PAD,), jnp.float32)
    p = _sc_spmv1(src, dst, features.reshape(N_NODES), zer1)
    x1full, x1g = _tc1(p.reshape(2, N_NODES, 1), W1, b1.reshape(1, 64))
    b2r = b2.reshape(1, 64)

    def layer_body(i, carry):
        xg, _ = carry
        ag = _sc_spmv64(src, dst, xg, zer)
        xg_next = _tc2(x1full, ag, W2, b2r)
        return (xg_next, ag)

    _, a3g = lax.fori_loop(0, 2, layer_body, (x1g, x1g))
    return _tc3(a3g, W3, b3.reshape(1, 128))


# speedup vs baseline: 1.2693x; 1.2693x over previous
import functools

import jax
import jax.numpy as jnp
from jax import lax
from jax.experimental import pallas as pl
from jax.experimental.pallas import tpu as pltpu
from jax.experimental.pallas import tpu_sc as plsc

N_NODES = 100000
N_EDGES = 1600000
GW = 16
NG = 4
K1 = 1000
K2 = 800
ROWS_PER_TILE = 6256
N_PAD = ROWS_PER_TILE * 16
LAST_ROWS = N_NODES - 15 * ROWS_PER_TILE


def _flush_acc_slice(acc, out, s):
    @pl.when(s < 15)
    def _():
        sl = pl.ds(s * ROWS_PER_TILE, ROWS_PER_TILE)
        pltpu.sync_copy(acc.at[sl], out.at[sl])

    @pl.when(s == 15)
    def _():
        sl = pl.ds(15 * ROWS_PER_TILE, LAST_ROWS)
        pltpu.sync_copy(acc.at[sl], out.at[sl])


def _zero_acc_slice(zer, acc, s):
    sl = pl.ds(s * ROWS_PER_TILE, ROWS_PER_TILE)
    pltpu.sync_copy(zer.at[sl], acc.at[sl])


def _edge_chunks(sr, dr, x_hbm, is2, id2, rw2, gA, gB, acc, base, k, n):
    def prefetch(i, b, g):
        off = base + i * k
        pltpu.sync_copy(sr.at[pl.ds(off, k)], is2.at[b])
        pltpu.sync_copy(dr.at[pl.ds(off, k)], id2.at[b])
        pltpu.async_copy(x_hbm.at[is2.at[b]], rw2.at[b], g)

    prefetch(0, 0, gA)

    def body(i, carry):
        par = i % 2

        @pl.when(par == 0)
        def _():
            @pl.when(i + 1 < n)
            def _():
                prefetch(i + 1, 1, gB)
            pltpu.make_async_copy(x_hbm.at[is2.at[0]], rw2.at[0], gA).wait()

        @pl.when(par == 1)
        def _():
            @pl.when(i + 1 < n)
            def _():
                prefetch(i + 1, 0, gA)
            pltpu.make_async_copy(x_hbm.at[is2.at[1]], rw2.at[1], gB).wait()

        pltpu.sync_copy(rw2.at[par], acc.at[id2.at[par]], add=True)
        return carry

    lax.fori_loop(0, n, body, 0)


_MESH = plsc.VectorSubcoreMesh(core_axis_name="c", subcore_axis_name="s")


@functools.partial(
    pl.kernel,
    mesh=_MESH,
    compiler_params=pltpu.CompilerParams(use_tc_tiling_on_sc=False),
    out_type=jax.ShapeDtypeStruct((2, N_NODES), jnp.float32),
    scratch_types=[
        pltpu.VMEM((2, K1), jnp.int32),
        pltpu.VMEM((2, K1), jnp.int32),
        pltpu.VMEM((2, K1), jnp.float32),
        pltpu.VMEM_SHARED((N_PAD,), jnp.float32),
        pltpu.SemaphoreType.DMA,
        pltpu.SemaphoreType.DMA,
    ],
)
def _sc_spmv1(src, dst, feats, zer1, pout,
              is2, id2, rw2, acc, gA, gB):
    c = lax.axis_index("c")
    s = lax.axis_index("s")
    _zero_acc_slice(zer1, acc, s)
    plsc.subcore_barrier()
    tid = c * 16 + s
    e_per_tile = N_EDGES // 32
    _edge_chunks(src, dst, feats, is2, id2, rw2, gA, gB, acc,
                 tid * e_per_tile, K1, e_per_tile // K1)
    plsc.subcore_barrier()
    _flush_acc_slice(acc, pout.at[c], s)


@functools.partial(
    pl.kernel,
    mesh=_MESH,
    compiler_params=pltpu.CompilerParams(use_tc_tiling_on_sc=False),
    out_type=jax.ShapeDtypeStruct((NG, N_NODES, GW), jnp.float32),
    scratch_types=[
        pltpu.VMEM((2, K2), jnp.int32),
        pltpu.VMEM((2, K2), jnp.int32),
        pltpu.VMEM((2, K2, GW), jnp.float32),
        pltpu.VMEM_SHARED((N_PAD, GW), jnp.float32),
        pltpu.SemaphoreType.DMA,
        pltpu.SemaphoreType.DMA,
    ],
)
def _sc_spmv64(src, dst, xin, zer, ost,
               is2, id2, rw2, acc, gA, gB):
    c = lax.axis_index("c")
    s = lax.axis_index("s")
    e_per_tile = N_EDGES // 16
    for p in range(2):
        g = 2 * p + c
        _zero_acc_slice(zer, acc, s)
        plsc.subcore_barrier()
        _edge_chunks(src, dst, xin.at[g], is2, id2, rw2, gA, gB, acc,
                     s * e_per_tile, K2, e_per_tile // K2)
        plsc.subcore_barrier()
        _flush_acc_slice(acc, ost.at[g], s)
        plsc.subcore_barrier()


NB = 2000


def _tc1_body(p, w1, b1, xfull, gout):
    a = p[0] + p[1]
    y = jnp.maximum(a * w1[...] + b1[...], 0.0)
    xfull[...] = y
    for i in range(NG):
        gout[i] = y[:, i * GW:(i + 1) * GW]


def _tc2_body(xf, ain, w2, b2, gout):
    acat = jnp.concatenate([ain[i] for i in range(NG)], axis=1)
    y = jnp.dot(acat, w2[...], preferred_element_type=jnp.float32)
    y = jnp.maximum(xf[...] + y + b2[...], 0.0)
    for i in range(NG):
        gout[i] = y[:, i * GW:(i + 1) * GW]


def _tc3_body(ain, w3, b3, o):
    acat = jnp.concatenate([ain[i] for i in range(NG)], axis=1)
    o[...] = jnp.dot(acat, w3[...], preferred_element_type=jnp.float32) + b3[...]


def _row_spec(cols):
    return pl.BlockSpec((NB, cols), lambda i: (i, 0))


def _full_spec(r, cols):
    return pl.BlockSpec((r, cols), lambda i: (0, 0))


_GRID = (N_NODES // NB,)

_P_SPEC = pl.BlockSpec((2, NB, 1), lambda i: (0, i, 0))
_G_SPEC = pl.BlockSpec((NG, NB, GW), lambda i: (0, i, 0))

_tc1 = pl.pallas_call(
    _tc1_body,
    grid=_GRID,
    in_specs=[_P_SPEC, _full_spec(1, 64), _full_spec(1, 64)],
    out_specs=[_row_spec(64), _G_SPEC],
    out_shape=[jax.ShapeDtypeStruct((N_NODES, 64), jnp.float32),
               jax.ShapeDtypeStruct((NG, N_NODES, GW), jnp.float32)],
)

_tc2 = pl.pallas_call(
    _tc2_body,
    grid=_GRID,
    in_specs=[_row_spec(64), _G_SPEC, _full_spec(64, 64), _full_spec(1, 64)],
    out_specs=_G_SPEC,
    out_shape=jax.ShapeDtypeStruct((NG, N_NODES, GW), jnp.float32),
)

_tc3 = pl.pallas_call(
    _tc3_body,
    grid=_GRID,
    in_specs=[_G_SPEC, _full_spec(64, 128), _full_spec(1, 128)],
    out_specs=_row_spec(128),
    out_shape=jax.ShapeDtypeStruct((N_NODES, 128), jnp.float32),
)


def kernel(features, edge_index, W1, b1, W2, b2, W3, b3):
    src = edge_index[0].astype(jnp.int32)
    dst = edge_index[1].astype(jnp.int32)
    zer = jnp.zeros((N_PAD, GW), jnp.float32)
    zer1 = jnp.zeros((N_PAD,), jnp.float32)
    p = _sc_spmv1(src, dst, features.reshape(N_NODES), zer1)
    x1full, x1g = _tc1(p.reshape(2, N_NODES, 1), W1, b1.reshape(1, 64))
    b2r = b2.reshape(1, 64)

    def layer_body(i, carry):
        xg, _ = carry
        ag = _sc_spmv64(src, dst, xg, zer)
        xg_next = _tc2(x1full, ag, W2, b2r)
        return (xg_next, ag)

    _, a3g = lax.fori_loop(0, 2, layer_body, (x1g, x1g))
    return _tc3(a3g, W3, b3.reshape(1, 128))
